# Initial kernel scaffold; baseline (speedup 1.0000x reference)
import jax
import jax.numpy as jnp
from jax.experimental import pallas as pl


def kernel(x, rots, trans, intrins, post_rots, post_trans, W, b):
    raise NotImplementedError("write your pallas kernel here")


# speedup vs baseline: 4.7544x; 4.7544x over previous
import functools

import jax
import jax.numpy as jnp
import numpy as np
from jax import lax
from jax.experimental import pallas as pl
from jax.experimental.pallas import tpu as pltpu
from jax.experimental.pallas import tpu_sc as plsc

_XB = (-51.2, 51.2, 0.8)
_YB = (-51.2, 51.2, 0.8)
_ZB = (-10.0, 10.0, 20.0)
_IMG_H, _IMG_W = 256, 704
_FH, _FW = 16, 44
_D = 41
_C = 64
_C_IN = 256
_NV = 24
_PIX = _FH * _FW
_NVOX = 128 * 128
_ACC = _NVOX + 128

_F_PAD = 112
_CTX0 = 48


def _tc_body(x_ref, w_ref, b_ref, wdep_ref, ctx_ref):
    xv = x_ref[0]
    feat = jnp.dot(w_ref[...], xv, preferred_element_type=jnp.float32)
    feat = feat + b_ref[:, 0:1]
    logits = feat[0:_D]
    m = jnp.max(logits, axis=0, keepdims=True)
    e = jnp.exp(logits - m)
    wdep_ref[0] = e / jnp.sum(e, axis=0, keepdims=True)
    ctx_ref[0] = feat[_CTX0:_F_PAD]


def _sc_body(ranks_hbm, wdep_hbm, ctx_hbm, bev_hbm,
             ctx_v, rk_v, wd_v, acc0, acc1):
    cid = lax.axis_index("c")
    sid = lax.axis_index("s")
    wid = sid * 2 + cid
    f0 = wid * 2

    def zero(i, carry):
        z = jnp.zeros((16,), jnp.float32)
        acc0[pl.ds(i * 16, 16)] = z
        acc1[pl.ds(i * 16, 16)] = z
        return carry

    lax.fori_loop(0, _ACC // 16, zero, 0, unroll=4)

    def view_body(vi, carry):
        v = lax.rem(vi + wid, _NV)
        pltpu.sync_copy(ctx_hbm.at[v, pl.ds(f0, 2), :], ctx_v)
        pltpu.sync_copy(ranks_hbm.at[v], rk_v)
        pltpu.sync_copy(wdep_hbm.at[v], wd_v)

        def pg_body(p, carry2):
            s = p * 16
            c0 = ctx_v[0, pl.ds(s, 16)]
            c1 = ctx_v[1, pl.ds(s, 16)]

            def d_body(d, carry3):
                r = rk_v[d, pl.ds(s, 16)]
                w = wd_v[d, pl.ds(s, 16)]
                plsc.addupdate_scatter(acc0, [r], w * c0)
                plsc.addupdate_scatter(acc1, [r], w * c1)
                return carry3

            return lax.fori_loop(0, _D, d_body, carry2)

        return lax.fori_loop(0, _PIX // 16, pg_body, carry)

    lax.fori_loop(0, _NV, view_body, 0)
    pltpu.sync_copy(acc0.at[pl.ds(0, _NVOX)], bev_hbm.at[f0])
    pltpu.sync_copy(acc1.at[pl.ds(0, _NVOX)], bev_hbm.at[f0 + 1])


@functools.lru_cache(maxsize=1)
def _make_sc():
    mesh = plsc.VectorSubcoreMesh(core_axis_name="c", subcore_axis_name="s")
    return pl.kernel(
        _sc_body,
        out_type=jax.ShapeDtypeStruct((_C, _NVOX), jnp.float32),
        mesh=mesh,
        compiler_params=pltpu.CompilerParams(needs_layout_passes=False),
        scratch_types=[
            pltpu.VMEM((2, _PIX), jnp.float32),
            pltpu.VMEM((_D, _PIX), jnp.int32),
            pltpu.VMEM((_D, _PIX), jnp.float32),
            pltpu.VMEM((_ACC,), jnp.float32),
            pltpu.VMEM((_ACC,), jnp.float32),
        ],
    )


def _tc_stage(xr, wp, bp):
    return pl.pallas_call(
        _tc_body,
        grid=(_NV,),
        in_specs=[
            pl.BlockSpec((1, _C_IN, _PIX), lambda v: (v, 0, 0)),
            pl.BlockSpec((_F_PAD, _C_IN), lambda v: (0, 0)),
            pl.BlockSpec((_F_PAD, 128), lambda v: (0, 0)),
        ],
        out_specs=[
            pl.BlockSpec((1, _D, _PIX), lambda v: (v, 0, 0)),
            pl.BlockSpec((1, _C, _PIX), lambda v: (v, 0, 0)),
        ],
        out_shape=[
            jax.ShapeDtypeStruct((_NV, _D, _PIX), jnp.float32),
            jax.ShapeDtypeStruct((_NV, _C, _PIX), jnp.float32),
        ],
    )(xr, wp, bp)


def _frustum_pts():
    ds = np.arange(4.0, 45.0, 1.0)
    d = ds.shape[0]
    dsb = jnp.broadcast_to(jnp.asarray(ds, jnp.float32)[:, None, None],
                           (d, _FH, _FW))
    xs = jnp.broadcast_to(
        jnp.linspace(0.0, _IMG_W - 1, _FW, dtype=jnp.float32)[None, None, :],
        (d, _FH, _FW))
    ys = jnp.broadcast_to(
        jnp.linspace(0.0, _IMG_H - 1, _FH, dtype=jnp.float32)[None, :, None],
        (d, _FH, _FW))
    return jnp.stack([xs, ys, dsb], -1)


def _voxel_ranks(rots, trans, intrins, post_rots, post_trans):
    frustum = _frustum_pts()
    pts = frustum[None, None] - post_trans[:, :, None, None, None, :]
    pts = jnp.einsum('bnij,bndhwj->bndhwi', jnp.linalg.inv(post_rots), pts)
    pts = jnp.concatenate([pts[..., :2] * pts[..., 2:3], pts[..., 2:3]],
                          axis=-1)
    comb = jnp.einsum('bnij,bnjk->bnik', rots, jnp.linalg.inv(intrins))
    pts = (jnp.einsum('bnij,bndhwj->bndhwi', comb, pts)
           + trans[:, :, None, None, None, :])
    rows = (_XB, _YB, _ZB)
    dx = jnp.array([r[2] for r in rows], jnp.float32)
    bx = jnp.array([r[0] + r[2] / 2.0 for r in rows], jnp.float32)
    g = pts.reshape(-1, 3)
    gi = ((g - (bx - dx / 2.0)) / dx).astype(jnp.int32)
    rank = gi[:, 0] + gi[:, 1] * 128 + gi[:, 2] * _NVOX
    kept = (rank >= 0) & (rank < _NVOX)
    lane = jnp.arange(rank.shape[0], dtype=jnp.int32) & 15
    return jnp.where(kept, rank, _NVOX + lane).reshape(_NV, _D, _PIX)


def kernel(x, rots, trans, intrins, post_rots, post_trans, W, b):
    f32 = jnp.float32
    xr = x.reshape(_NV, _C_IN, _PIX)

    wp = jnp.zeros((_F_PAD, _C_IN), f32)
    wp = wp.at[0:_D].set(W[0:_D]).at[_CTX0:_F_PAD].set(W[_D:])
    bp = jnp.zeros((_F_PAD,), f32)
    bp = bp.at[0:_D].set(b[0:_D]).at[_CTX0:_F_PAD].set(b[_D:])
    bp = jnp.broadcast_to(bp[:, None], (_F_PAD, 128))

    ranks = _voxel_ranks(rots, trans, intrins, post_rots, post_trans)
    wdep, ctx = _tc_stage(xr, wp, bp)
    bev = _make_sc()(ranks, wdep, ctx)
    return bev.reshape(1, _C, 128, 128)


# speedup vs baseline: 4.8881x; 1.0281x over previous
import functools

import jax
import jax.numpy as jnp
import numpy as np
from jax import lax
from jax.experimental import pallas as pl
from jax.experimental.pallas import tpu as pltpu
from jax.experimental.pallas import tpu_sc as plsc

_XB = (-51.2, 51.2, 0.8)
_YB = (-51.2, 51.2, 0.8)
_ZB = (-10.0, 10.0, 20.0)
_IMG_H, _IMG_W = 256, 704
_FH, _FW = 16, 44
_D = 41
_C = 64
_C_IN = 256
_NV = 24
_PIX = _FH * _FW
_NVOX = 128 * 128
_ACC = _NVOX + 128

_F_PAD = 112
_CTX0 = 48


def _tc_body(x_ref, w_ref, b_ref, wdep_ref, ctx_ref):
    xv = x_ref[0]
    feat = jnp.dot(w_ref[...], xv, preferred_element_type=jnp.float32)
    feat = feat + b_ref[:, 0:1]
    logits = feat[0:_D]
    m = jnp.max(logits, axis=0, keepdims=True)
    e = jnp.exp(logits - m)
    wdep_ref[0] = e / jnp.sum(e, axis=0, keepdims=True)
    ctx_ref[0] = feat[_CTX0:_F_PAD]


def _sc_body(ranks_hbm, wdep_hbm, ctx_hbm, bev_hbm,
             ctx_v, rk_v, wd_v, acc0, acc1):
    cid = lax.axis_index("c")
    sid = lax.axis_index("s")
    wid = sid * 2 + cid
    f0 = wid * 2

    def zero(i, carry):
        z = jnp.zeros((16,), jnp.float32)
        acc0[pl.ds(i * 16, 16)] = z
        acc1[pl.ds(i * 16, 16)] = z
        return carry

    lax.fori_loop(0, _ACC // 16, zero, 0, unroll=4)

    def view_body(vi, carry):
        v = lax.rem(vi + wid, _NV)
        pltpu.sync_copy(ctx_hbm.at[v, pl.ds(f0, 2), :], ctx_v)
        pltpu.sync_copy(ranks_hbm.at[v], rk_v)
        pltpu.sync_copy(wdep_hbm.at[v], wd_v)

        def pg_body(p, carry2):
            s = p * 16
            c0 = ctx_v[0, pl.ds(s, 16)]
            c1 = ctx_v[1, pl.ds(s, 16)]

            def d_body(d, carry3):
                r = rk_v[d, pl.ds(s, 16)]
                w = wd_v[d, pl.ds(s, 16)]
                plsc.addupdate_scatter(acc0, [r], w * c0)
                plsc.addupdate_scatter(acc1, [r], w * c1)
                return carry3

            return lax.fori_loop(0, _D, d_body, carry2, unroll=_D)

        return lax.fori_loop(0, _PIX // 16, pg_body, carry)

    lax.fori_loop(0, _NV, view_body, 0)
    pltpu.sync_copy(acc0.at[pl.ds(0, _NVOX)], bev_hbm.at[f0])
    pltpu.sync_copy(acc1.at[pl.ds(0, _NVOX)], bev_hbm.at[f0 + 1])


@functools.lru_cache(maxsize=1)
def _make_sc():
    mesh = plsc.VectorSubcoreMesh(core_axis_name="c", subcore_axis_name="s")
    return pl.kernel(
        _sc_body,
        out_type=jax.ShapeDtypeStruct((_C, _NVOX), jnp.float32),
        mesh=mesh,
        compiler_params=pltpu.CompilerParams(needs_layout_passes=False),
        scratch_types=[
            pltpu.VMEM((2, _PIX), jnp.float32),
            pltpu.VMEM((_D, _PIX), jnp.int32),
            pltpu.VMEM((_D, _PIX), jnp.float32),
            pltpu.VMEM((_ACC,), jnp.float32),
            pltpu.VMEM((_ACC,), jnp.float32),
        ],
    )


def _tc_stage(xr, wp, bp):
    return pl.pallas_call(
        _tc_body,
        grid=(_NV,),
        in_specs=[
            pl.BlockSpec((1, _C_IN, _PIX), lambda v: (v, 0, 0)),
            pl.BlockSpec((_F_PAD, _C_IN), lambda v: (0, 0)),
            pl.BlockSpec((_F_PAD, 128), lambda v: (0, 0)),
        ],
        out_specs=[
            pl.BlockSpec((1, _D, _PIX), lambda v: (v, 0, 0)),
            pl.BlockSpec((1, _C, _PIX), lambda v: (v, 0, 0)),
        ],
        out_shape=[
            jax.ShapeDtypeStruct((_NV, _D, _PIX), jnp.float32),
            jax.ShapeDtypeStruct((_NV, _C, _PIX), jnp.float32),
        ],
    )(xr, wp, bp)


def _frustum_pts():
    ds = np.arange(4.0, 45.0, 1.0)
    d = ds.shape[0]
    dsb = jnp.broadcast_to(jnp.asarray(ds, jnp.float32)[:, None, None],
                           (d, _FH, _FW))
    xs = jnp.broadcast_to(
        jnp.linspace(0.0, _IMG_W - 1, _FW, dtype=jnp.float32)[None, None, :],
        (d, _FH, _FW))
    ys = jnp.broadcast_to(
        jnp.linspace(0.0, _IMG_H - 1, _FH, dtype=jnp.float32)[None, :, None],
        (d, _FH, _FW))
    return jnp.stack([xs, ys, dsb], -1)


def _voxel_ranks(rots, trans, intrins, post_rots, post_trans):
    frustum = _frustum_pts()
    pts = frustum[None, None] - post_trans[:, :, None, None, None, :]
    pts = jnp.einsum('bnij,bndhwj->bndhwi', jnp.linalg.inv(post_rots), pts)
    pts = jnp.concatenate([pts[..., :2] * pts[..., 2:3], pts[..., 2:3]],
                          axis=-1)
    comb = jnp.einsum('bnij,bnjk->bnik', rots, jnp.linalg.inv(intrins))
    pts = (jnp.einsum('bnij,bndhwj->bndhwi', comb, pts)
           + trans[:, :, None, None, None, :])
    rows = (_XB, _YB, _ZB)
    dx = jnp.array([r[2] for r in rows], jnp.float32)
    bx = jnp.array([r[0] + r[2] / 2.0 for r in rows], jnp.float32)
    g = pts.reshape(-1, 3)
    gi = ((g - (bx - dx / 2.0)) / dx).astype(jnp.int32)
    rank = gi[:, 0] + gi[:, 1] * 128 + gi[:, 2] * _NVOX
    kept = (rank >= 0) & (rank < _NVOX)
    lane = jnp.arange(rank.shape[0], dtype=jnp.int32) & 15
    return jnp.where(kept, rank, _NVOX + lane).reshape(_NV, _D, _PIX)


def kernel(x, rots, trans, intrins, post_rots, post_trans, W, b):
    f32 = jnp.float32
    xr = x.reshape(_NV, _C_IN, _PIX)

    wp = jnp.zeros((_F_PAD, _C_IN), f32)
    wp = wp.at[0:_D].set(W[0:_D]).at[_CTX0:_F_PAD].set(W[_D:])
    bp = jnp.zeros((_F_PAD,), f32)
    bp = bp.at[0:_D].set(b[0:_D]).at[_CTX0:_F_PAD].set(b[_D:])
    bp = jnp.broadcast_to(bp[:, None], (_F_PAD, 128))

    ranks = _voxel_ranks(rots, trans, intrins, post_rots, post_trans)
    wdep, ctx = _tc_stage(xr, wp, bp)
    bev = _make_sc()(ranks, wdep, ctx)
    return bev.reshape(1, _C, 128, 128)


# speedup vs baseline: 4.8893x; 1.0002x over previous
import functools

import jax
import jax.numpy as jnp
import numpy as np
from jax import lax
from jax.experimental import pallas as pl
from jax.experimental.pallas import tpu as pltpu
from jax.experimental.pallas import tpu_sc as plsc

_XB = (-51.2, 51.2, 0.8)
_YB = (-51.2, 51.2, 0.8)
_ZB = (-10.0, 10.0, 20.0)
_IMG_H, _IMG_W = 256, 704
_FH, _FW = 16, 44
_D = 41
_C = 64
_C_IN = 256
_NV = 24
_PIX = _FH * _FW
_NVOX = 128 * 128
_ACC = _NVOX + 128

_F_PAD = 112
_CTX0 = 48


def _tc_body(x_ref, w_ref, b_ref, rk_ref, packed_ref, ctx_ref):
    xv = x_ref[0]
    feat = jnp.dot(w_ref[...], xv, preferred_element_type=jnp.float32)
    feat = feat + b_ref[:, 0:1]
    logits = feat[0:_D]
    m = jnp.max(logits, axis=0, keepdims=True)
    e = jnp.exp(logits - m)
    depth = e / jnp.sum(e, axis=0, keepdims=True)
    db = depth.astype(jnp.bfloat16)
    bits = lax.bitcast_convert_type(db, jnp.uint16).astype(jnp.int32) << 16
    packed_ref[0] = bits | rk_ref[0]
    ctx_ref[0] = feat[_CTX0:_F_PAD]


def _sc_body(packed_hbm, ctx_hbm, bev_hbm, ctx_v, pk_v, acc0, acc1):
    cid = lax.axis_index("c")
    sid = lax.axis_index("s")
    wid = sid * 2 + cid
    f0 = wid * 2

    def zero(i, carry):
        z = jnp.zeros((16,), jnp.float32)
        acc0[pl.ds(i * 16, 16)] = z
        acc1[pl.ds(i * 16, 16)] = z
        return carry

    lax.fori_loop(0, _ACC // 16, zero, 0, unroll=4)

    lo_mask = jnp.full((16,), 0xFFFF, jnp.int32)
    hi_mask = jnp.full((16,), -65536, jnp.int32)

    def view_body(vi, carry):
        v = lax.rem(vi + wid, _NV)
        pltpu.sync_copy(ctx_hbm.at[v, pl.ds(f0, 2), :], ctx_v)
        pltpu.sync_copy(packed_hbm.at[v], pk_v)

        def pg_body(p, carry2):
            s = p * 16
            c0 = ctx_v[0, pl.ds(s, 16)]
            c1 = ctx_v[1, pl.ds(s, 16)]

            def d_body(d, carry3):
                word = pk_v[d, pl.ds(s, 16)]
                r = word & lo_mask
                w = plsc.bitcast(word & hi_mask, jnp.float32)
                plsc.addupdate_scatter(acc0, [r], w * c0)
                plsc.addupdate_scatter(acc1, [r], w * c1)
                return carry3

            return lax.fori_loop(0, _D, d_body, carry2, unroll=_D)

        return lax.fori_loop(0, _PIX // 16, pg_body, carry)

    lax.fori_loop(0, _NV, view_body, 0)
    pltpu.sync_copy(acc0.at[pl.ds(0, _NVOX)], bev_hbm.at[f0])
    pltpu.sync_copy(acc1.at[pl.ds(0, _NVOX)], bev_hbm.at[f0 + 1])


@functools.lru_cache(maxsize=1)
def _make_sc():
    mesh = plsc.VectorSubcoreMesh(core_axis_name="c", subcore_axis_name="s")
    return pl.kernel(
        _sc_body,
        out_type=jax.ShapeDtypeStruct((_C, _NVOX), jnp.float32),
        mesh=mesh,
        compiler_params=pltpu.CompilerParams(needs_layout_passes=False),
        scratch_types=[
            pltpu.VMEM((2, _PIX), jnp.float32),
            pltpu.VMEM((_D, _PIX), jnp.int32),
            pltpu.VMEM((_ACC,), jnp.float32),
            pltpu.VMEM((_ACC,), jnp.float32),
        ],
    )


def _tc_stage(xr, wp, bp, ranks):
    return pl.pallas_call(
        _tc_body,
        grid=(_NV,),
        in_specs=[
            pl.BlockSpec((1, _C_IN, _PIX), lambda v: (v, 0, 0)),
            pl.BlockSpec((_F_PAD, _C_IN), lambda v: (0, 0)),
            pl.BlockSpec((_F_PAD, 128), lambda v: (0, 0)),
            pl.BlockSpec((1, _D, _PIX), lambda v: (v, 0, 0)),
        ],
        out_specs=[
            pl.BlockSpec((1, _D, _PIX), lambda v: (v, 0, 0)),
            pl.BlockSpec((1, _C, _PIX), lambda v: (v, 0, 0)),
        ],
        out_shape=[
            jax.ShapeDtypeStruct((_NV, _D, _PIX), jnp.int32),
            jax.ShapeDtypeStruct((_NV, _C, _PIX), jnp.float32),
        ],
    )(xr, wp, bp, ranks)


def _frustum_pts():
    ds = np.arange(4.0, 45.0, 1.0)
    d = ds.shape[0]
    dsb = jnp.broadcast_to(jnp.asarray(ds, jnp.float32)[:, None, None],
                           (d, _FH, _FW))
    xs = jnp.broadcast_to(
        jnp.linspace(0.0, _IMG_W - 1, _FW, dtype=jnp.float32)[None, None, :],
        (d, _FH, _FW))
    ys = jnp.broadcast_to(
        jnp.linspace(0.0, _IMG_H - 1, _FH, dtype=jnp.float32)[None, :, None],
        (d, _FH, _FW))
    return jnp.stack([xs, ys, dsb], -1)


def _voxel_ranks(rots, trans, intrins, post_rots, post_trans):
    frustum = _frustum_pts()
    pts = frustum[None, None] - post_trans[:, :, None, None, None, :]
    pts = jnp.einsum('bnij,bndhwj->bndhwi', jnp.linalg.inv(post_rots), pts)
    pts = jnp.concatenate([pts[..., :2] * pts[..., 2:3], pts[..., 2:3]],
                          axis=-1)
    comb = jnp.einsum('bnij,bnjk->bnik', rots, jnp.linalg.inv(intrins))
    pts = (jnp.einsum('bnij,bndhwj->bndhwi', comb, pts)
           + trans[:, :, None, None, None, :])
    rows = (_XB, _YB, _ZB)
    dx = jnp.array([r[2] for r in rows], jnp.float32)
    bx = jnp.array([r[0] + r[2] / 2.0 for r in rows], jnp.float32)
    g = pts.reshape(-1, 3)
    gi = ((g - (bx - dx / 2.0)) / dx).astype(jnp.int32)
    rank = gi[:, 0] + gi[:, 1] * 128 + gi[:, 2] * _NVOX
    kept = (rank >= 0) & (rank < _NVOX)
    lane = jnp.arange(rank.shape[0], dtype=jnp.int32) & 15
    return jnp.where(kept, rank, _NVOX + lane).reshape(_NV, _D, _PIX)


def kernel(x, rots, trans, intrins, post_rots, post_trans, W, b):
    f32 = jnp.float32
    xr = x.reshape(_NV, _C_IN, _PIX)

    wp = jnp.zeros((_F_PAD, _C_IN), f32)
    wp = wp.at[0:_D].set(W[0:_D]).at[_CTX0:_F_PAD].set(W[_D:])
    bp = jnp.zeros((_F_PAD,), f32)
    bp = bp.at[0:_D].set(b[0:_D]).at[_CTX0:_F_PAD].set(b[_D:])
    bp = jnp.broadcast_to(bp[:, None], (_F_PAD, 128))

    ranks = _voxel_ranks(rots, trans, intrins, post_rots, post_trans)
    packed, ctx = _tc_stage(xr, wp, bp, ranks)
    bev = _make_sc()(packed, ctx)
    return bev.reshape(1, _C, 128, 128)


# speedup vs baseline: 5.8104x; 1.1884x over previous
import functools

import jax
import jax.numpy as jnp
import numpy as np
from jax import lax
from jax.experimental import pallas as pl
from jax.experimental.pallas import tpu as pltpu
from jax.experimental.pallas import tpu_sc as plsc

_XB = (-51.2, 51.2, 0.8)
_YB = (-51.2, 51.2, 0.8)
_ZB = (-10.0, 10.0, 20.0)
_IMG_H, _IMG_W = 256, 704
_FH, _FW = 16, 44
_D = 41
_C = 64
_C_IN = 256
_NV = 24
_PIX = _FH * _FW
_NVOX = 128 * 128
_ACC = _NVOX + 128

_F_PAD = 112
_CTX0 = 48


def _tc_body(x_ref, w_ref, b_ref, rk_ref, packed_ref, ctx_ref):
    xv = x_ref[0]
    feat = jnp.dot(w_ref[...], xv, preferred_element_type=jnp.float32)
    feat = feat + b_ref[:, 0:1]
    logits = feat[0:_D]
    m = jnp.max(logits, axis=0, keepdims=True)
    e = jnp.exp(logits - m)
    depth = e / jnp.sum(e, axis=0, keepdims=True)
    db = depth.astype(jnp.bfloat16)
    bits = lax.bitcast_convert_type(db, jnp.uint16).astype(jnp.int32) << 16
    packed_ref[0] = bits | rk_ref[0]
    ctx_ref[0] = feat[_CTX0:_F_PAD]


def _sc_body(packed_hbm, ctx_hbm, bev_hbm, ctx_v, pk_v, acc0, acc1):
    cid = lax.axis_index("c")
    sid = lax.axis_index("s")
    wid = sid * 2 + cid
    f0 = wid * 2

    def zero(i, carry):
        z = jnp.zeros((16,), jnp.float32)
        acc0[pl.ds(i * 16, 16)] = z
        acc1[pl.ds(i * 16, 16)] = z
        return carry

    lax.fori_loop(0, _ACC // 16, zero, 0, unroll=4)

    lo_mask = jnp.full((16,), 0xFFFF, jnp.int32)
    hi_mask = jnp.full((16,), -65536, jnp.int32)

    def view_body(vi, carry):
        v = lax.rem(vi + wid, _NV)
        pltpu.sync_copy(ctx_hbm.at[v, pl.ds(f0, 2), :], ctx_v)
        pltpu.sync_copy(packed_hbm.at[v], pk_v)

        def pg_body(p, carry2):
            s = p * 32
            c0a = ctx_v[0, pl.ds(s, 16)]
            c1a = ctx_v[1, pl.ds(s, 16)]
            c0b = ctx_v[0, pl.ds(s + 16, 16)]
            c1b = ctx_v[1, pl.ds(s + 16, 16)]

            def d_body(d, carry3):
                wa = pk_v[d, pl.ds(s, 16)]
                wb = pk_v[d, pl.ds(s + 16, 16)]
                ra = wa & lo_mask
                rb = wb & lo_mask
                va = plsc.bitcast(wa & hi_mask, jnp.float32)
                vb = plsc.bitcast(wb & hi_mask, jnp.float32)
                plsc.addupdate_scatter(acc0, [ra], va * c0a)
                plsc.addupdate_scatter(acc1, [ra], va * c1a)
                plsc.addupdate_scatter(acc0, [rb], vb * c0b)
                plsc.addupdate_scatter(acc1, [rb], vb * c1b)
                return carry3

            return lax.fori_loop(0, _D, d_body, carry2, unroll=_D)

        return lax.fori_loop(0, _PIX // 32, pg_body, carry)

    lax.fori_loop(0, _NV, view_body, 0)
    pltpu.sync_copy(acc0.at[pl.ds(0, _NVOX)], bev_hbm.at[f0])
    pltpu.sync_copy(acc1.at[pl.ds(0, _NVOX)], bev_hbm.at[f0 + 1])


@functools.lru_cache(maxsize=1)
def _make_sc():
    mesh = plsc.VectorSubcoreMesh(core_axis_name="c", subcore_axis_name="s")
    return pl.kernel(
        _sc_body,
        out_type=jax.ShapeDtypeStruct((_C, _NVOX), jnp.float32),
        mesh=mesh,
        compiler_params=pltpu.CompilerParams(needs_layout_passes=False),
        scratch_types=[
            pltpu.VMEM((2, _PIX), jnp.float32),
            pltpu.VMEM((_D, _PIX), jnp.int32),
            pltpu.VMEM((_ACC,), jnp.float32),
            pltpu.VMEM((_ACC,), jnp.float32),
        ],
    )


def _tc_stage(xr, wp, bp, ranks):
    return pl.pallas_call(
        _tc_body,
        grid=(_NV,),
        in_specs=[
            pl.BlockSpec((1, _C_IN, _PIX), lambda v: (v, 0, 0)),
            pl.BlockSpec((_F_PAD, _C_IN), lambda v: (0, 0)),
            pl.BlockSpec((_F_PAD, 128), lambda v: (0, 0)),
            pl.BlockSpec((1, _D, _PIX), lambda v: (v, 0, 0)),
        ],
        out_specs=[
            pl.BlockSpec((1, _D, _PIX), lambda v: (v, 0, 0)),
            pl.BlockSpec((1, _C, _PIX), lambda v: (v, 0, 0)),
        ],
        out_shape=[
            jax.ShapeDtypeStruct((_NV, _D, _PIX), jnp.int32),
            jax.ShapeDtypeStruct((_NV, _C, _PIX), jnp.float32),
        ],
    )(xr, wp, bp, ranks)


def _frustum_pts():
    ds = np.arange(4.0, 45.0, 1.0)
    d = ds.shape[0]
    dsb = jnp.broadcast_to(jnp.asarray(ds, jnp.float32)[:, None, None],
                           (d, _FH, _FW))
    xs = jnp.broadcast_to(
        jnp.linspace(0.0, _IMG_W - 1, _FW, dtype=jnp.float32)[None, None, :],
        (d, _FH, _FW))
    ys = jnp.broadcast_to(
        jnp.linspace(0.0, _IMG_H - 1, _FH, dtype=jnp.float32)[None, :, None],
        (d, _FH, _FW))
    return jnp.stack([xs, ys, dsb], -1)


def _voxel_ranks(rots, trans, intrins, post_rots, post_trans):
    frustum = _frustum_pts()
    pts = frustum[None, None] - post_trans[:, :, None, None, None, :]
    pts = jnp.einsum('bnij,bndhwj->bndhwi', jnp.linalg.inv(post_rots), pts)
    pts = jnp.concatenate([pts[..., :2] * pts[..., 2:3], pts[..., 2:3]],
                          axis=-1)
    comb = jnp.einsum('bnij,bnjk->bnik', rots, jnp.linalg.inv(intrins))
    pts = (jnp.einsum('bnij,bndhwj->bndhwi', comb, pts)
           + trans[:, :, None, None, None, :])
    rows = (_XB, _YB, _ZB)
    dx = jnp.array([r[2] for r in rows], jnp.float32)
    bx = jnp.array([r[0] + r[2] / 2.0 for r in rows], jnp.float32)
    g = pts.reshape(-1, 3)
    gi = ((g - (bx - dx / 2.0)) / dx).astype(jnp.int32)
    rank = gi[:, 0] + gi[:, 1] * 128 + gi[:, 2] * _NVOX
    kept = (rank >= 0) & (rank < _NVOX)
    lane = jnp.arange(rank.shape[0], dtype=jnp.int32) & 15
    return jnp.where(kept, rank, _NVOX + lane).reshape(_NV, _D, _PIX)


def kernel(x, rots, trans, intrins, post_rots, post_trans, W, b):
    f32 = jnp.float32
    xr = x.reshape(_NV, _C_IN, _PIX)

    wp = jnp.zeros((_F_PAD, _C_IN), f32)
    wp = wp.at[0:_D].set(W[0:_D]).at[_CTX0:_F_PAD].set(W[_D:])
    bp = jnp.zeros((_F_PAD,), f32)
    bp = bp.at[0:_D].set(b[0:_D]).at[_CTX0:_F_PAD].set(b[_D:])
    bp = jnp.broadcast_to(bp[:, None], (_F_PAD, 128))

    ranks = _voxel_ranks(rots, trans, intrins, post_rots, post_trans)
    packed, ctx = _tc_stage(xr, wp, bp, ranks)
    bev = _make_sc()(packed, ctx)
    return bev.reshape(1, _C, 128, 128)


# speedup vs baseline: 6.5448x; 1.1264x over previous
import functools

import jax
import jax.numpy as jnp
import numpy as np
from jax import lax
from jax.experimental import pallas as pl
from jax.experimental.pallas import tpu as pltpu
from jax.experimental.pallas import tpu_sc as plsc

_XB = (-51.2, 51.2, 0.8)
_YB = (-51.2, 51.2, 0.8)
_ZB = (-10.0, 10.0, 20.0)
_IMG_H, _IMG_W = 256, 704
_FH, _FW = 16, 44
_D = 41
_C = 64
_C_IN = 256
_NV = 24
_PIX = _FH * _FW
_NVOX = 128 * 128
_ACC = _NVOX + 128

_F_PAD = 112
_CTX0 = 48


def _tc_body(x_ref, w_ref, b_ref, rk_ref, packed_ref, ctx_ref):
    xv = x_ref[0]
    feat = jnp.dot(w_ref[...], xv, preferred_element_type=jnp.float32)
    feat = feat + b_ref[:, 0:1]
    logits = feat[0:_D]
    m = jnp.max(logits, axis=0, keepdims=True)
    e = jnp.exp(logits - m)
    depth = e / jnp.sum(e, axis=0, keepdims=True)
    db = depth.astype(jnp.bfloat16)
    bits = lax.bitcast_convert_type(db, jnp.uint16).astype(jnp.int32) << 16
    packed_ref[0] = bits | rk_ref[0]
    ctx_ref[0] = feat[_CTX0:_F_PAD]


def _sc_body(packed_hbm, ctx_hbm, bev_hbm, ctx_v, pk_v, acc0, acc1):
    cid = lax.axis_index("c")
    sid = lax.axis_index("s")
    wid = sid * 2 + cid
    f0 = wid * 2

    def zero(i, carry):
        z = jnp.zeros((16,), jnp.float32)
        acc0[pl.ds(i * 16, 16)] = z
        acc1[pl.ds(i * 16, 16)] = z
        return carry

    lax.fori_loop(0, _ACC // 16, zero, 0, unroll=4)

    lo_mask = jnp.full((16,), 0xFFFF, jnp.int32)
    hi_mask = jnp.full((16,), -65536, jnp.int32)

    def view_body(vi, carry):
        v = lax.rem(vi + wid, _NV)
        pltpu.sync_copy(ctx_hbm.at[v, pl.ds(f0, 2), :], ctx_v)
        pltpu.sync_copy(packed_hbm.at[v], pk_v)

        def pg_body(p, carry2):
            s = p * 64
            cs = [(ctx_v[0, pl.ds(s + 16 * i, 16)],
                   ctx_v[1, pl.ds(s + 16 * i, 16)]) for i in range(4)]

            def d_body(d, carry3):
                ws = [pk_v[d, pl.ds(s + 16 * i, 16)] for i in range(4)]
                rs = [w & lo_mask for w in ws]
                vs = [plsc.bitcast(w & hi_mask, jnp.float32) for w in ws]
                for i in range(4):
                    plsc.addupdate_scatter(acc0, [rs[i]], vs[i] * cs[i][0])
                    plsc.addupdate_scatter(acc1, [rs[i]], vs[i] * cs[i][1])
                return carry3

            return lax.fori_loop(0, _D, d_body, carry2, unroll=_D)

        return lax.fori_loop(0, _PIX // 64, pg_body, carry)

    lax.fori_loop(0, _NV, view_body, 0)
    pltpu.sync_copy(acc0.at[pl.ds(0, _NVOX)], bev_hbm.at[f0])
    pltpu.sync_copy(acc1.at[pl.ds(0, _NVOX)], bev_hbm.at[f0 + 1])


@functools.lru_cache(maxsize=1)
def _make_sc():
    mesh = plsc.VectorSubcoreMesh(core_axis_name="c", subcore_axis_name="s")
    return pl.kernel(
        _sc_body,
        out_type=jax.ShapeDtypeStruct((_C, _NVOX), jnp.float32),
        mesh=mesh,
        compiler_params=pltpu.CompilerParams(needs_layout_passes=False),
        scratch_types=[
            pltpu.VMEM((2, _PIX), jnp.float32),
            pltpu.VMEM((_D, _PIX), jnp.int32),
            pltpu.VMEM((_ACC,), jnp.float32),
            pltpu.VMEM((_ACC,), jnp.float32),
        ],
    )


def _tc_stage(xr, wp, bp, ranks):
    return pl.pallas_call(
        _tc_body,
        grid=(_NV,),
        in_specs=[
            pl.BlockSpec((1, _C_IN, _PIX), lambda v: (v, 0, 0)),
            pl.BlockSpec((_F_PAD, _C_IN), lambda v: (0, 0)),
            pl.BlockSpec((_F_PAD, 128), lambda v: (0, 0)),
            pl.BlockSpec((1, _D, _PIX), lambda v: (v, 0, 0)),
        ],
        out_specs=[
            pl.BlockSpec((1, _D, _PIX), lambda v: (v, 0, 0)),
            pl.BlockSpec((1, _C, _PIX), lambda v: (v, 0, 0)),
        ],
        out_shape=[
            jax.ShapeDtypeStruct((_NV, _D, _PIX), jnp.int32),
            jax.ShapeDtypeStruct((_NV, _C, _PIX), jnp.float32),
        ],
    )(xr, wp, bp, ranks)


def _frustum_pts():
    ds = np.arange(4.0, 45.0, 1.0)
    d = ds.shape[0]
    dsb = jnp.broadcast_to(jnp.asarray(ds, jnp.float32)[:, None, None],
                           (d, _FH, _FW))
    xs = jnp.broadcast_to(
        jnp.linspace(0.0, _IMG_W - 1, _FW, dtype=jnp.float32)[None, None, :],
        (d, _FH, _FW))
    ys = jnp.broadcast_to(
        jnp.linspace(0.0, _IMG_H - 1, _FH, dtype=jnp.float32)[None, :, None],
        (d, _FH, _FW))
    return jnp.stack([xs, ys, dsb], -1)


def _voxel_ranks(rots, trans, intrins, post_rots, post_trans):
    frustum = _frustum_pts()
    pts = frustum[None, None] - post_trans[:, :, None, None, None, :]
    pts = jnp.einsum('bnij,bndhwj->bndhwi', jnp.linalg.inv(post_rots), pts)
    pts = jnp.concatenate([pts[..., :2] * pts[..., 2:3], pts[..., 2:3]],
                          axis=-1)
    comb = jnp.einsum('bnij,bnjk->bnik', rots, jnp.linalg.inv(intrins))
    pts = (jnp.einsum('bnij,bndhwj->bndhwi', comb, pts)
           + trans[:, :, None, None, None, :])
    rows = (_XB, _YB, _ZB)
    dx = jnp.array([r[2] for r in rows], jnp.float32)
    bx = jnp.array([r[0] + r[2] / 2.0 for r in rows], jnp.float32)
    g = pts.reshape(-1, 3)
    gi = ((g - (bx - dx / 2.0)) / dx).astype(jnp.int32)
    rank = gi[:, 0] + gi[:, 1] * 128 + gi[:, 2] * _NVOX
    kept = (rank >= 0) & (rank < _NVOX)
    lane = jnp.arange(rank.shape[0], dtype=jnp.int32) & 15
    return jnp.where(kept, rank, _NVOX + lane).reshape(_NV, _D, _PIX)


def kernel(x, rots, trans, intrins, post_rots, post_trans, W, b):
    f32 = jnp.float32
    xr = x.reshape(_NV, _C_IN, _PIX)

    wp = jnp.zeros((_F_PAD, _C_IN), f32)
    wp = wp.at[0:_D].set(W[0:_D]).at[_CTX0:_F_PAD].set(W[_D:])
    bp = jnp.zeros((_F_PAD,), f32)
    bp = bp.at[0:_D].set(b[0:_D]).at[_CTX0:_F_PAD].set(b[_D:])
    bp = jnp.broadcast_to(bp[:, None], (_F_PAD, 128))

    ranks = _voxel_ranks(rots, trans, intrins, post_rots, post_trans)
    packed, ctx = _tc_stage(xr, wp, bp, ranks)
    bev = _make_sc()(packed, ctx)
    return bev.reshape(1, _C, 128, 128)


# speedup vs baseline: 7.0220x; 1.0729x over previous
import functools

import jax
import jax.numpy as jnp
import numpy as np
from jax import lax
from jax.experimental import pallas as pl
from jax.experimental.pallas import tpu as pltpu
from jax.experimental.pallas import tpu_sc as plsc

_XB = (-51.2, 51.2, 0.8)
_YB = (-51.2, 51.2, 0.8)
_ZB = (-10.0, 10.0, 20.0)
_IMG_H, _IMG_W = 256, 704
_FH, _FW = 16, 44
_D = 41
_C = 64
_C_IN = 256
_NV = 24
_PIX = _FH * _FW
_NVOX = 128 * 128
_ACC = _NVOX + 128

_F_PAD = 112
_CTX0 = 48


def _tc_body(x_ref, w_ref, b_ref, rk_ref, packed_ref, ctx_ref):
    xv = x_ref[0]
    feat = jnp.dot(w_ref[...], xv, preferred_element_type=jnp.float32)
    feat = feat + b_ref[:, 0:1]
    logits = feat[0:_D]
    m = jnp.max(logits, axis=0, keepdims=True)
    e = jnp.exp(logits - m)
    depth = e / jnp.sum(e, axis=0, keepdims=True)
    db = depth.astype(jnp.bfloat16)
    bits = lax.bitcast_convert_type(db, jnp.uint16).astype(jnp.int32) << 16
    packed_ref[0] = bits | rk_ref[0]
    ctx_ref[0] = feat[_CTX0:_F_PAD]


def _sc_body(packed_hbm, ctx_hbm, bev_hbm, ctx_v, pk_v, acc0, acc1):
    cid = lax.axis_index("c")
    sid = lax.axis_index("s")
    wid = sid * 2 + cid
    f0 = wid * 2

    def zero(i, carry):
        z = jnp.zeros((16,), jnp.float32)
        acc0[pl.ds(i * 16, 16)] = z
        acc1[pl.ds(i * 16, 16)] = z
        return carry

    lax.fori_loop(0, _ACC // 16, zero, 0, unroll=4)

    lo_mask = jnp.full((16,), 0xFFFF, jnp.int32)
    hi_mask = jnp.full((16,), -65536, jnp.int32)

    def view_body(vi, carry):
        v = lax.rem(vi + wid, _NV)
        pltpu.sync_copy(ctx_hbm.at[v, pl.ds(f0, 2), :], ctx_v)
        pltpu.sync_copy(packed_hbm.at[v], pk_v)

        def pg_body(p, carry2):
            s = p * 176
            cs = [(ctx_v[0, pl.ds(s + 16 * i, 16)],
                   ctx_v[1, pl.ds(s + 16 * i, 16)]) for i in range(11)]

            def d_body(d, carry3):
                ws = [pk_v[d, pl.ds(s + 16 * i, 16)] for i in range(11)]
                rs = [w & lo_mask for w in ws]
                vs = [plsc.bitcast(w & hi_mask, jnp.float32) for w in ws]
                for i in range(11):
                    plsc.addupdate_scatter(acc0, [rs[i]], vs[i] * cs[i][0])
                    plsc.addupdate_scatter(acc1, [rs[i]], vs[i] * cs[i][1])
                return carry3

            return lax.fori_loop(0, _D, d_body, carry2, unroll=_D)

        return lax.fori_loop(0, _PIX // 176, pg_body, carry)

    lax.fori_loop(0, _NV, view_body, 0)
    pltpu.sync_copy(acc0.at[pl.ds(0, _NVOX)], bev_hbm.at[f0])
    pltpu.sync_copy(acc1.at[pl.ds(0, _NVOX)], bev_hbm.at[f0 + 1])


@functools.lru_cache(maxsize=1)
def _make_sc():
    mesh = plsc.VectorSubcoreMesh(core_axis_name="c", subcore_axis_name="s")
    return pl.kernel(
        _sc_body,
        out_type=jax.ShapeDtypeStruct((_C, _NVOX), jnp.float32),
        mesh=mesh,
        compiler_params=pltpu.CompilerParams(needs_layout_passes=False),
        scratch_types=[
            pltpu.VMEM((2, _PIX), jnp.float32),
            pltpu.VMEM((_D, _PIX), jnp.int32),
            pltpu.VMEM((_ACC,), jnp.float32),
            pltpu.VMEM((_ACC,), jnp.float32),
        ],
    )


def _tc_stage(xr, wp, bp, ranks):
    return pl.pallas_call(
        _tc_body,
        grid=(_NV,),
        in_specs=[
            pl.BlockSpec((1, _C_IN, _PIX), lambda v: (v, 0, 0)),
            pl.BlockSpec((_F_PAD, _C_IN), lambda v: (0, 0)),
            pl.BlockSpec((_F_PAD, 128), lambda v: (0, 0)),
            pl.BlockSpec((1, _D, _PIX), lambda v: (v, 0, 0)),
        ],
        out_specs=[
            pl.BlockSpec((1, _D, _PIX), lambda v: (v, 0, 0)),
            pl.BlockSpec((1, _C, _PIX), lambda v: (v, 0, 0)),
        ],
        out_shape=[
            jax.ShapeDtypeStruct((_NV, _D, _PIX), jnp.int32),
            jax.ShapeDtypeStruct((_NV, _C, _PIX), jnp.float32),
        ],
    )(xr, wp, bp, ranks)


def _frustum_pts():
    ds = np.arange(4.0, 45.0, 1.0)
    d = ds.shape[0]
    dsb = jnp.broadcast_to(jnp.asarray(ds, jnp.float32)[:, None, None],
                           (d, _FH, _FW))
    xs = jnp.broadcast_to(
        jnp.linspace(0.0, _IMG_W - 1, _FW, dtype=jnp.float32)[None, None, :],
        (d, _FH, _FW))
    ys = jnp.broadcast_to(
        jnp.linspace(0.0, _IMG_H - 1, _FH, dtype=jnp.float32)[None, :, None],
        (d, _FH, _FW))
    return jnp.stack([xs, ys, dsb], -1)


def _voxel_ranks(rots, trans, intrins, post_rots, post_trans):
    frustum = _frustum_pts()
    pts = frustum[None, None] - post_trans[:, :, None, None, None, :]
    pts = jnp.einsum('bnij,bndhwj->bndhwi', jnp.linalg.inv(post_rots), pts)
    pts = jnp.concatenate([pts[..., :2] * pts[..., 2:3], pts[..., 2:3]],
                          axis=-1)
    comb = jnp.einsum('bnij,bnjk->bnik', rots, jnp.linalg.inv(intrins))
    pts = (jnp.einsum('bnij,bndhwj->bndhwi', comb, pts)
           + trans[:, :, None, None, None, :])
    rows = (_XB, _YB, _ZB)
    dx = jnp.array([r[2] for r in rows], jnp.float32)
    bx = jnp.array([r[0] + r[2] / 2.0 for r in rows], jnp.float32)
    g = pts.reshape(-1, 3)
    gi = ((g - (bx - dx / 2.0)) / dx).astype(jnp.int32)
    rank = gi[:, 0] + gi[:, 1] * 128 + gi[:, 2] * _NVOX
    kept = (rank >= 0) & (rank < _NVOX)
    lane = jnp.arange(rank.shape[0], dtype=jnp.int32) & 15
    return jnp.where(kept, rank, _NVOX + lane).reshape(_NV, _D, _PIX)


def kernel(x, rots, trans, intrins, post_rots, post_trans, W, b):
    f32 = jnp.float32
    xr = x.reshape(_NV, _C_IN, _PIX)

    wp = jnp.zeros((_F_PAD, _C_IN), f32)
    wp = wp.at[0:_D].set(W[0:_D]).at[_CTX0:_F_PAD].set(W[_D:])
    bp = jnp.zeros((_F_PAD,), f32)
    bp = bp.at[0:_D].set(b[0:_D]).at[_CTX0:_F_PAD].set(b[_D:])
    bp = jnp.broadcast_to(bp[:, None], (_F_PAD, 128))

    ranks = _voxel_ranks(rots, trans, intrins, post_rots, post_trans)
    packed, ctx = _tc_stage(xr, wp, bp, ranks)
    bev = _make_sc()(packed, ctx)
    return bev.reshape(1, _C, 128, 128)
